# baseline (speedup 1.0000x reference)
import jax
import jax.numpy as jnp
from jax.experimental import pallas as pl
from jax.experimental.pallas import tpu as pltpu


def _copy_block(x_ref, o_ref):
    o_ref[...] = x_ref[...]


def kernel(X):
    b, n, f = X.shape
    total = b * n * f
    width = 1024
    rows = total // width
    flat = X.reshape(rows, width)
    grid = (4,)
    blk = rows // 4
    out = pl.pallas_call(
        _copy_block,
        grid=grid,
        in_specs=[pl.BlockSpec((blk, width), lambda i: (i, 0))],
        out_specs=pl.BlockSpec((blk, width), lambda i: (i, 0)),
        out_shape=jax.ShapeDtypeStruct((rows, width), X.dtype),
    )(flat)
    return out.reshape(b, n, f)
